# baseline (speedup 1.0000x reference)
import functools

import jax
import jax.numpy as jnp
from jax import lax
from jax.experimental import pallas as pl
from jax.experimental.pallas import tpu as pltpu
from jax.experimental.pallas import tpu_sc as plsc

N = 64
H = 512
W = 512

NUM_CORES = 2
NUM_SUBCORES = 16
NUM_WORKERS = NUM_CORES * NUM_SUBCORES
N_PER_WORKER = N // NUM_WORKERS

ROWS = 16
SLABS_PER_N = H // ROWS
TOTAL_SLABS = N_PER_WORKER * SLABS_PER_N
SLOTS = 6
LANES = 16
SEGS = W // LANES
VECS = ROWS * SEGS
UNROLL = 8


def _sc_body(frames, rd_h, out, rv, in0, in1, sin, sout):
    wid = lax.axis_index("s") * NUM_CORES + lax.axis_index("c")

    pltpu.sync_copy(rd_h, rv)

    one = jnp.full((LANES,), 1.0, jnp.float32)
    zero = jnp.full((LANES,), 0.0, jnp.float32)
    ws = []
    for j in range(N_PER_WORKER):
        n = wid * N_PER_WORKER + j
        r = rv[n, :]
        d = rv[N + n, :]
        ws.append((jnp.where(d > 0, one - r, jnp.where(d < 0, r, one)),
                   jnp.where(d > 0, r, jnp.where(d < 0, one - r, zero))))

    def slab_addr(g):
        n = wid * N_PER_WORKER + g // SLABS_PER_N
        row0 = (g % SLABS_PER_N) * ROWS
        return n, row0

    def issue_in(g, s):
        n, row0 = slab_addr(g)
        pltpu.async_copy(frames.at[n, 0, pl.ds(row0, ROWS), :],
                         in0.at[s], sin.at[s])
        pltpu.async_copy(frames.at[n, 1, pl.ds(row0, ROWS), :],
                         in1.at[s], sin.at[s])

    issue_in(0, 0)
    issue_in(1, 1)

    def step(g, carry):
        s = lax.rem(g, SLOTS)
        n, row0 = slab_addr(g)
        pltpu.make_async_copy(frames.at[n, 0, pl.ds(row0, ROWS), :],
                              in0.at[s], sin.at[s]).wait()
        pltpu.make_async_copy(frames.at[n, 1, pl.ds(row0, ROWS), :],
                              in1.at[s], sin.at[s]).wait()

        sel = g >= SLABS_PER_N
        av = jnp.where(sel, ws[1][0], ws[0][0])
        bv = jnp.where(sel, ws[1][1], ws[0][1])

        @plsc.parallel_loop(0, VECS, step=1, unroll=UNROLL)
        def blend(i):
            row = i // SEGS
            c = (i % SEGS) * LANES
            x0 = in0[s, row, pl.ds(c, LANES)]
            x1 = in1[s, row, pl.ds(c, LANES)]
            in0[s, row, pl.ds(c, LANES)] = av * x0 + bv * x1

        pltpu.async_copy(in0.at[s], out.at[n, 0, pl.ds(row0, ROWS), :],
                         sout.at[s])

        s2 = lax.rem(g + 2, SLOTS)
        n2, row2 = slab_addr(lax.min(g + 2, TOTAL_SLABS - 1))

        @pl.when(jnp.logical_and(g >= SLOTS - 2, g + 2 < TOTAL_SLABS))
        def _():
            pltpu.make_async_copy(in0.at[s2],
                                  out.at[n2, 0, pl.ds(row2, ROWS), :],
                                  sout.at[s2]).wait()

        @pl.when(g + 2 < TOTAL_SLABS)
        def _():
            pltpu.async_copy(frames.at[n2, 0, pl.ds(row2, ROWS), :],
                             in0.at[s2], sin.at[s2])
            pltpu.async_copy(frames.at[n2, 1, pl.ds(row2, ROWS), :],
                             in1.at[s2], sin.at[s2])
        return carry

    lax.fori_loop(0, TOTAL_SLABS, step, 0)

    n_last, _ = slab_addr(0)
    for s in range(SLOTS):
        pltpu.make_async_copy(in0.at[s],
                              out.at[n_last, 0, pl.ds(0, ROWS), :],
                              sout.at[s]).wait()


_sc_call = functools.partial(
    pl.kernel,
    mesh=plsc.VectorSubcoreMesh(core_axis_name="c", subcore_axis_name="s"),
    out_type=jax.ShapeDtypeStruct((N, 1, H, W), jnp.float32),
    compiler_params=pltpu.CompilerParams(use_tc_tiling_on_sc=True),
    scratch_types=[
        pltpu.VMEM((2 * N, LANES), jnp.float32),
        pltpu.VMEM((SLOTS, ROWS, W), jnp.float32),
        pltpu.VMEM((SLOTS, ROWS, W), jnp.float32),
        pltpu.SemaphoreType.DMA((SLOTS,)),
        pltpu.SemaphoreType.DMA((SLOTS,)),
    ],
)(_sc_body)


def kernel(exist_frames, ratio, direction):
    rd = jnp.concatenate([ratio, direction], axis=0)
    rd_b = jnp.broadcast_to(rd, (2 * N, LANES))
    return _sc_call(exist_frames, rd_b)


# speedup vs baseline: 1.0836x; 1.0836x over previous
import functools

import jax
import jax.numpy as jnp
from jax import lax
from jax.experimental import pallas as pl
from jax.experimental.pallas import tpu as pltpu
from jax.experimental.pallas import tpu_sc as plsc

N = 64
H = 512
W = 512

NUM_CORES = 2
NUM_SUBCORES = 16
NUM_WORKERS = NUM_CORES * NUM_SUBCORES
N_PER_WORKER = N // NUM_WORKERS

ROWS = 32
SLABS_PER_N = H // ROWS
TOTAL_SLABS = N_PER_WORKER * SLABS_PER_N
SLOTS = 3
LANES = 16
SEGS = W // LANES
VECS = ROWS * SEGS
UNROLL = 8


def _sc_body(frames, rd_h, out, rv, in0, in1, sin, sout):
    wid = lax.axis_index("s") * NUM_CORES + lax.axis_index("c")

    pltpu.sync_copy(rd_h, rv)

    one = jnp.full((LANES,), 1.0, jnp.float32)
    zero = jnp.full((LANES,), 0.0, jnp.float32)
    ws = []
    for j in range(N_PER_WORKER):
        n = wid * N_PER_WORKER + j
        r = rv[n, :]
        d = rv[N + n, :]
        ws.append((jnp.where(d > 0, one - r, jnp.where(d < 0, r, one)),
                   jnp.where(d > 0, r, jnp.where(d < 0, one - r, zero))))

    def slab_addr(g):
        n = wid * N_PER_WORKER + g // SLABS_PER_N
        row0 = (g % SLABS_PER_N) * ROWS
        return n, row0

    def issue_in(g, s):
        n, row0 = slab_addr(g)
        pltpu.async_copy(frames.at[n, 0, pl.ds(row0, ROWS), :],
                         in0.at[s], sin.at[s])
        pltpu.async_copy(frames.at[n, 1, pl.ds(row0, ROWS), :],
                         in1.at[s], sin.at[s])

    issue_in(0, 0)
    issue_in(1, 1)

    def step(g, carry):
        s = lax.rem(g, SLOTS)
        n, row0 = slab_addr(g)
        pltpu.make_async_copy(frames.at[n, 0, pl.ds(row0, ROWS), :],
                              in0.at[s], sin.at[s]).wait()
        pltpu.make_async_copy(frames.at[n, 1, pl.ds(row0, ROWS), :],
                              in1.at[s], sin.at[s]).wait()

        sel = g >= SLABS_PER_N
        av = jnp.where(sel, ws[1][0], ws[0][0])
        bv = jnp.where(sel, ws[1][1], ws[0][1])

        @plsc.parallel_loop(0, VECS, step=1, unroll=UNROLL)
        def blend(i):
            row = i // SEGS
            c = (i % SEGS) * LANES
            x0 = in0[s, row, pl.ds(c, LANES)]
            x1 = in1[s, row, pl.ds(c, LANES)]
            in0[s, row, pl.ds(c, LANES)] = av * x0 + bv * x1

        pltpu.async_copy(in0.at[s], out.at[n, 0, pl.ds(row0, ROWS), :],
                         sout.at[s])

        s2 = lax.rem(g + 2, SLOTS)
        n2, row2 = slab_addr(lax.min(g + 2, TOTAL_SLABS - 1))

        @pl.when(jnp.logical_and(g >= SLOTS - 2, g + 2 < TOTAL_SLABS))
        def _():
            pltpu.make_async_copy(in0.at[s2],
                                  out.at[n2, 0, pl.ds(row2, ROWS), :],
                                  sout.at[s2]).wait()

        @pl.when(g + 2 < TOTAL_SLABS)
        def _():
            pltpu.async_copy(frames.at[n2, 0, pl.ds(row2, ROWS), :],
                             in0.at[s2], sin.at[s2])
            pltpu.async_copy(frames.at[n2, 1, pl.ds(row2, ROWS), :],
                             in1.at[s2], sin.at[s2])
        return carry

    lax.fori_loop(0, TOTAL_SLABS, step, 0)

    n_last, _ = slab_addr(0)
    for s in range(SLOTS):
        pltpu.make_async_copy(in0.at[s],
                              out.at[n_last, 0, pl.ds(0, ROWS), :],
                              sout.at[s]).wait()


_sc_call = functools.partial(
    pl.kernel,
    mesh=plsc.VectorSubcoreMesh(core_axis_name="c", subcore_axis_name="s"),
    out_type=jax.ShapeDtypeStruct((N, 1, H, W), jnp.float32),
    compiler_params=pltpu.CompilerParams(use_tc_tiling_on_sc=True),
    scratch_types=[
        pltpu.VMEM((2 * N, LANES), jnp.float32),
        pltpu.VMEM((SLOTS, ROWS, W), jnp.float32),
        pltpu.VMEM((SLOTS, ROWS, W), jnp.float32),
        pltpu.SemaphoreType.DMA((SLOTS,)),
        pltpu.SemaphoreType.DMA((SLOTS,)),
    ],
)(_sc_body)


def kernel(exist_frames, ratio, direction):
    rd = jnp.concatenate([ratio, direction], axis=0)
    rd_b = jnp.broadcast_to(rd, (2 * N, LANES))
    return _sc_call(exist_frames, rd_b)
